# baseline (speedup 1.0000x reference)
import jax
import jax.numpy as jnp
from jax import lax
from jax.experimental import pallas as pl
from jax.experimental.pallas import tpu as pltpu

_N, _E, _D, _DE = 10000, 320000, 128, 16
_G = 5
_NBLK = _N // _G
_EBLK = _E // _G
_S = 8
_ECH = _EBLK // _S


def _bce(logits, labels):
    return (jnp.maximum(logits, 0.0) - logits * labels
            + jnp.log1p(jnp.exp(-jnp.abs(logits))))


def _fused(nf_ref, nlab_ref, eft_ref, elab_ref, nb_ref, eb_ref,
           wn1_ref, bn1_ref, wn2_ref, bn2_ref,
           we1_ref, we2_ref, be2_ref,
           nlog_ref, elog_ref, nsum_ref, esum_ref):
    i = pl.program_id(0)

    nh = jnp.maximum(
        jnp.dot(nf_ref[...].reshape(_NBLK, _D), wn1_ref[...],
                preferred_element_type=jnp.float32)
        + bn1_ref[...], 0.0)
    nlogit = (lax.dot_general(wn2_ref[...], nh, (((1,), (1,)), ((), ())),
                              preferred_element_type=jnp.float32)
              + bn2_ref[...])
    nlog_ref[...] = nlogit.reshape(1, 1, _NBLK)

    eh = jnp.maximum(
        jnp.dot(we1_ref[...], eft_ref[...], preferred_element_type=jnp.float32),
        0.0)
    w2 = we2_ref[...]
    elogit = jnp.concatenate(
        [jnp.dot(w2, eh[:, s * _ECH:(s + 1) * _ECH],
                 preferred_element_type=jnp.float32) for s in range(_S)],
        axis=0) + be2_ref[...]
    elog_ref[...] = elogit.reshape(1, _S, _ECH)

    @pl.when(i == 0)
    def _init():
        nsum_ref[...] = jnp.zeros_like(nsum_ref)
        esum_ref[...] = jnp.zeros_like(esum_ref)

    nlab = nlab_ref[...].reshape(1, _NBLK)
    elab = elab_ref[...].reshape(_S, _ECH)
    nsum_ref[...] += jnp.sum(_bce(nlogit, nlab)).reshape(1, 1)
    esum_ref[...] += jnp.sum(_bce(elogit, elab)).reshape(1, 1)

    @pl.when(i == _G - 1)
    def _norm():
        dn = jnp.max(nb_ref[...]).astype(jnp.float32) + 1.0
        de = jnp.max(eb_ref[...]).astype(jnp.float32) + 1.0
        nsum_ref[...] = nsum_ref[...] / dn
        esum_ref[...] = esum_ref[...] / de


def kernel(node_feat, edge_feat, node_label, edge_label, node_batch,
           edge_batch, Wn1, bn1, Wn2, bn2, We1, be1, We2, be2):
    eft = jnp.concatenate([edge_feat.T, jnp.ones((1, _E), jnp.float32)])
    elab = edge_label.reshape(_G, _S, _ECH)
    nlab = node_label.reshape(_G, 1, _NBLK)
    nf3 = node_feat.reshape(_G, _NBLK, _D)
    nb = node_batch.reshape(_G, 1, _NBLK)
    eb = edge_batch.reshape(_G, _S, _ECH)

    row3 = lambda i: (i, 0, 0)
    col = lambda i: (0, i)
    fixed = lambda i: (0, 0)
    last3 = lambda i: (_G - 1, 0, 0)
    full = lambda a: pl.BlockSpec(a.shape, fixed)

    nlog, elog, nsum, esum = pl.pallas_call(
        _fused,
        grid=(_G,),
        in_specs=[
            pl.BlockSpec((1, _NBLK, _D), row3),
            pl.BlockSpec((1, 1, _NBLK), row3),
            pl.BlockSpec((_DE + 1, _EBLK), col),
            pl.BlockSpec((1, _S, _ECH), row3),
            pl.BlockSpec((1, 1, _NBLK), last3),
            pl.BlockSpec((1, _S, _ECH), last3),
            full(Wn1),
            pl.BlockSpec((1, _D), fixed),
            pl.BlockSpec((1, _D), fixed),
            pl.BlockSpec((1, 1), fixed),
            pl.BlockSpec((_DE, _DE + 1), fixed),
            pl.BlockSpec((1, _DE), fixed),
            pl.BlockSpec((1, 1), fixed),
        ],
        out_specs=[
            pl.BlockSpec((1, 1, _NBLK), row3),
            pl.BlockSpec((1, _S, _ECH), row3),
            pl.BlockSpec((1, 1), fixed),
            pl.BlockSpec((1, 1), fixed),
        ],
        out_shape=[
            jax.ShapeDtypeStruct((_G, 1, _NBLK), jnp.float32),
            jax.ShapeDtypeStruct((_G, _S, _ECH), jnp.float32),
            jax.ShapeDtypeStruct((1, 1), jnp.float32),
            jax.ShapeDtypeStruct((1, 1), jnp.float32),
        ],
        compiler_params=pltpu.CompilerParams(
            dimension_semantics=("arbitrary",)),
    )(nf3, nlab, eft, elab, nb, eb,
      Wn1, bn1.reshape(1, _D), Wn2.T, bn2.reshape(1, 1),
      jnp.concatenate([We1.T, be1.reshape(_DE, 1)], axis=1),
      We2.T, be2.reshape(1, 1))

    return (nlog.reshape(_N), elog.reshape(_E),
            nsum.reshape(()), esum.reshape(()))


# speedup vs baseline: 1.5928x; 1.5928x over previous
import jax
import jax.numpy as jnp
from jax import lax
from jax.experimental import pallas as pl
from jax.experimental.pallas import tpu as pltpu

_N, _E, _D, _DE = 10000, 320000, 128, 16
_G = 5
_NBLK = _N // _G
_EBLK = _E // _G
_S = 8
_ECH = _EBLK // _S


def _bce(logits, labels):
    return (jnp.maximum(logits, 0.0) - logits * labels
            + jnp.log1p(jnp.exp(-jnp.abs(logits))))


def _fused(nf_ref, nlab_ref, eft_ref, elab_ref, nb_ref, eb_ref,
           wn1_ref, bn1_ref, wn2_ref, bn2_ref,
           we1_ref, be1_ref, we2_ref, be2_ref,
           nlog_ref, elog_ref, nsum_ref, esum_ref):
    i = pl.program_id(0)

    nh = jnp.maximum(
        jnp.dot(nf_ref[...].reshape(_NBLK, _D), wn1_ref[...],
                preferred_element_type=jnp.float32)
        + bn1_ref[...], 0.0)
    nlogit = (lax.dot_general(wn2_ref[...], nh, (((1,), (1,)), ((), ())),
                              preferred_element_type=jnp.float32)
              + bn2_ref[...])
    nlog_ref[...] = nlogit.reshape(1, 1, _NBLK)

    eh = jnp.maximum(
        jnp.dot(we1_ref[...], eft_ref[...], preferred_element_type=jnp.float32)
        + be1_ref[...], 0.0)
    w2 = we2_ref[...]
    elogit = jnp.concatenate(
        [jnp.dot(w2, eh[:, s * _ECH:(s + 1) * _ECH],
                 preferred_element_type=jnp.float32) for s in range(_S)],
        axis=0) + be2_ref[...]
    elog_ref[...] = elogit.reshape(1, _S, _ECH)

    @pl.when(i == 0)
    def _init():
        nsum_ref[...] = jnp.zeros_like(nsum_ref)
        esum_ref[...] = jnp.zeros_like(esum_ref)

    nlab = nlab_ref[...].reshape(1, _NBLK)
    elab = elab_ref[...].reshape(_S, _ECH)
    nsum_ref[...] += jnp.sum(_bce(nlogit, nlab)).reshape(1, 1)
    esum_ref[...] += jnp.sum(_bce(elogit, elab)).reshape(1, 1)

    @pl.when(i == _G - 1)
    def _norm():
        dn = jnp.max(nb_ref[...]).astype(jnp.float32) + 1.0
        de = jnp.max(eb_ref[...]).astype(jnp.float32) + 1.0
        nsum_ref[...] = nsum_ref[...] / dn
        esum_ref[...] = esum_ref[...] / de


def kernel(node_feat, edge_feat, node_label, edge_label, node_batch,
           edge_batch, Wn1, bn1, Wn2, bn2, We1, be1, We2, be2):
    eft = edge_feat.T
    elab = edge_label.reshape(_G, _S, _ECH)
    nlab = node_label.reshape(_G, 1, _NBLK)
    nf3 = node_feat.reshape(_G, _NBLK, _D)
    nb = node_batch.reshape(_G, 1, _NBLK)
    eb = edge_batch.reshape(_G, _S, _ECH)

    row3 = lambda i: (i, 0, 0)
    col = lambda i: (0, i)
    fixed = lambda i: (0, 0)
    last3 = lambda i: (_G - 1, 0, 0)
    full = lambda a: pl.BlockSpec(a.shape, fixed)

    nlog, elog, nsum, esum = pl.pallas_call(
        _fused,
        grid=(_G,),
        in_specs=[
            pl.BlockSpec((1, _NBLK, _D), row3),
            pl.BlockSpec((1, 1, _NBLK), row3),
            pl.BlockSpec((_DE, _EBLK), col),
            pl.BlockSpec((1, _S, _ECH), row3),
            pl.BlockSpec((1, 1, _NBLK), last3),
            pl.BlockSpec((1, _S, _ECH), last3),
            full(Wn1),
            pl.BlockSpec((1, _D), fixed),
            pl.BlockSpec((1, _D), fixed),
            pl.BlockSpec((1, 1), fixed),
            pl.BlockSpec((_DE, _DE), fixed),
            pl.BlockSpec((_DE, 1), fixed),
            pl.BlockSpec((1, _DE), fixed),
            pl.BlockSpec((1, 1), fixed),
        ],
        out_specs=[
            pl.BlockSpec((1, 1, _NBLK), row3),
            pl.BlockSpec((1, _S, _ECH), row3),
            pl.BlockSpec((1, 1), fixed),
            pl.BlockSpec((1, 1), fixed),
        ],
        out_shape=[
            jax.ShapeDtypeStruct((_G, 1, _NBLK), jnp.float32),
            jax.ShapeDtypeStruct((_G, _S, _ECH), jnp.float32),
            jax.ShapeDtypeStruct((1, 1), jnp.float32),
            jax.ShapeDtypeStruct((1, 1), jnp.float32),
        ],
        compiler_params=pltpu.CompilerParams(
            dimension_semantics=("arbitrary",)),
    )(nf3, nlab, eft, elab, nb, eb,
      Wn1, bn1.reshape(1, _D), Wn2.T, bn2.reshape(1, 1),
      We1.T, be1.reshape(_DE, 1), We2.T, be2.reshape(1, 1))

    return (nlog.reshape(_N), elog.reshape(_E),
            nsum.reshape(()), esum.reshape(()))
